# baseline (speedup 1.0000x reference)
import functools

import jax
import jax.numpy as jnp
from jax import lax
from jax.experimental import pallas as pl
from jax.experimental.pallas import tpu as pltpu
from jax.experimental.pallas import tpu_sc as plsc

B = 4096
F = 26
V = 100000
D = 32
NUM = 13

_NC = 2
_NS = 16
_NW = _NC * _NS
_FD = F * D
_ROWS_PER_W = _FD // _NW
_LANES = 16


def _sc_gather_t(tt, catT):
    mesh = plsc.VectorSubcoreMesh(core_axis_name="c", subcore_axis_name="s")

    @functools.partial(
        pl.kernel,
        out_type=jax.ShapeDtypeStruct((_FD, B), jnp.float32),
        mesh=mesh,
        scratch_types=[
            pltpu.VMEM((V,), jnp.float32),
            pltpu.VMEM((B,), jnp.int32),
            pltpu.VMEM((B,), jnp.float32),
            pltpu.VMEM((B,), jnp.float32),
            pltpu.SemaphoreType.DMA,
            pltpu.SemaphoreType.DMA,
            pltpu.SemaphoreType.DMA,
        ],
        compiler_params=pltpu.CompilerParams(needs_layout_passes=False),
    )
    def gather_kernel(tt_hbm, catT_hbm, out_hbm, row_v, idx_v, o0_v, o1_v,
                      row_sem, osem0, osem1):
        wid = lax.axis_index("s") * _NC + lax.axis_index("c")
        base = wid * _ROWS_PER_W
        osems = [osem0, osem1]
        obufs = [o0_v, o1_v]
        pltpu.sync_copy(catT_hbm.at[base // D], idx_v)
        row_copy = pltpu.async_copy(tt_hbm.at[base], row_v, row_sem)
        out_copies = [None, None]
        for j in range(_ROWS_PER_W):
            r = base + j
            if j > 0:
                @pl.when((base + j) // D != (base + j - 1) // D)
                def _():
                    pltpu.sync_copy(catT_hbm.at[r // D], idx_v)
            row_copy.wait()
            if out_copies[j % 2] is not None:
                out_copies[j % 2].wait()

            ob = obufs[j % 2]

            def body(i, carry):
                o = pl.multiple_of(i * _LANES, _LANES)
                iv = idx_v[pl.ds(o, _LANES)]
                ob[pl.ds(o, _LANES)] = plsc.load_gather(row_v, [iv])
                return carry

            lax.fori_loop(0, B // _LANES, body, 0, unroll=8)
            if j + 1 < _ROWS_PER_W:
                row_copy = pltpu.async_copy(tt_hbm.at[r + 1], row_v, row_sem)
            out_copies[j % 2] = pltpu.async_copy(
                ob, out_hbm.at[r], osems[j % 2])
        for c in out_copies:
            if c is not None:
                c.wait()

    return gather_kernel(tt, catT)


_BN = 512


def _mlp_kernel(embT_ref, numT_ref, w1e_ref, w1n_ref, b1_ref, w2_ref, b2_ref,
                w3_ref, b3_ref, w4_ref, b4_ref, out_ref):
    x1 = jnp.dot(w1e_ref[...], embT_ref[...],
                 preferred_element_type=jnp.float32)
    x1 = x1 + jnp.dot(w1n_ref[...], numT_ref[...],
                      preferred_element_type=jnp.float32)
    h1 = jnp.maximum(x1 + b1_ref[...], 0.0)
    h2 = jnp.maximum(
        jnp.dot(w2_ref[...], h1, preferred_element_type=jnp.float32)
        + b2_ref[...], 0.0)
    h3 = jnp.maximum(
        jnp.dot(w3_ref[...], h2, preferred_element_type=jnp.float32)
        + b3_ref[...], 0.0)
    out_ref[...] = (
        jnp.sum(h3 * w4_ref[...], axis=0, keepdims=True) + b4_ref[...])


def _mlp(embT, numT_pad, W1eT, W1nT, b1c, W2T, b2c, W3T, b3c, w4c, b4):
    grid = (B // _BN,)
    full = lambda shape: pl.BlockSpec(shape, lambda i: (0, 0))
    return pl.pallas_call(
        _mlp_kernel,
        grid=grid,
        in_specs=[
            pl.BlockSpec((_FD, _BN), lambda i: (0, i)),
            pl.BlockSpec((16, _BN), lambda i: (0, i)),
            full((512, _FD)),
            full((512, 16)),
            full((512, 1)),
            full((256, 512)),
            full((256, 1)),
            full((128, 256)),
            full((128, 1)),
            full((128, 1)),
            full((1, 1)),
        ],
        out_specs=pl.BlockSpec((1, _BN), lambda i: (0, i)),
        out_shape=jax.ShapeDtypeStruct((1, B), jnp.float32),
        compiler_params=pltpu.CompilerParams(
            dimension_semantics=("arbitrary",),
        ),
    )(embT, numT_pad, W1eT, W1nT, b1c, W2T, b2c, W3T, b3c, w4c, b4)


def kernel(numerical_features, cat_features, tables, W1, b1, W2, b2, W3, b3,
           W4, b4):
    tt = jnp.transpose(tables, (0, 2, 1)).reshape(_FD, V)
    catT = cat_features.astype(jnp.int32).T

    embT = _sc_gather_t(tt, catT)

    numT_pad = jnp.pad(numerical_features.T, ((0, 16 - NUM), (0, 0)))
    W1eT = W1[:_FD].T
    W1nT = jnp.pad(W1[_FD:], ((0, 16 - NUM), (0, 0))).T
    outT = _mlp(embT, numT_pad, W1eT, W1nT,
                b1.reshape(-1, 1), W2.T, b2.reshape(-1, 1),
                W3.T, b3.reshape(-1, 1), W4.reshape(-1, 1),
                b4.reshape(1, 1))
    return outT.reshape(B, 1)


# speedup vs baseline: 1.3072x; 1.3072x over previous
import functools

import jax
import jax.numpy as jnp
from jax import lax
from jax.experimental import pallas as pl
from jax.experimental.pallas import tpu as pltpu
from jax.experimental.pallas import tpu_sc as plsc

B = 4096
F = 26
V = 100000
D = 32
NUM = 13

_NC = 2
_NS = 16
_NW = _NC * _NS
_FD = F * D
_ROWS_PER_W = _FD // _NW
_LANES = 16


def _sc_gather_t(tt, catT):
    mesh = plsc.VectorSubcoreMesh(core_axis_name="c", subcore_axis_name="s")

    @functools.partial(
        pl.kernel,
        out_type=jax.ShapeDtypeStruct((_FD, B), jnp.float32),
        mesh=mesh,
        scratch_types=[
            pltpu.VMEM((V,), jnp.float32),
            pltpu.VMEM((B,), jnp.int32),
            pltpu.VMEM((B,), jnp.float32),
            pltpu.VMEM((B,), jnp.float32),
            pltpu.SemaphoreType.DMA,
            pltpu.SemaphoreType.DMA,
            pltpu.SemaphoreType.DMA,
        ],
        compiler_params=pltpu.CompilerParams(needs_layout_passes=False),
    )
    def gather_kernel(tt_hbm, catT_hbm, out_hbm, row_v, idx_v, o0_v, o1_v,
                      row_sem, osem0, osem1):
        wid = lax.axis_index("s") * _NC + lax.axis_index("c")
        base = wid * _ROWS_PER_W
        osems = [osem0, osem1]
        obufs = [o0_v, o1_v]
        pltpu.sync_copy(catT_hbm.at[base // D], idx_v)
        for j in range(_ROWS_PER_W):
            r = base + j
            if j > 0:
                @pl.when((base + j) // D != (base + j - 1) // D)
                def _():
                    pltpu.sync_copy(catT_hbm.at[r // D], idx_v)
            pltpu.sync_copy(tt_hbm.at[r], row_v)
            ob = obufs[j % 2]

            @functools.partial(plsc.parallel_loop, 0, B // _LANES, unroll=1)
            def _(i):
                o = pl.multiple_of(i * _LANES, _LANES)
                iv = idx_v[pl.ds(o, _LANES)]
                ob[pl.ds(o, _LANES)] = plsc.load_gather(row_v, [iv])

            pltpu.sync_copy(ob, out_hbm.at[r])

    return gather_kernel(tt, catT)


_BN = 512


def _mlp_kernel(embT_ref, numT_ref, w1e_ref, w1n_ref, b1_ref, w2_ref, b2_ref,
                w3_ref, b3_ref, w4_ref, b4_ref, out_ref):
    x1 = jnp.dot(w1e_ref[...], embT_ref[...],
                 preferred_element_type=jnp.float32)
    x1 = x1 + jnp.dot(w1n_ref[...], numT_ref[...],
                      preferred_element_type=jnp.float32)
    h1 = jnp.maximum(x1 + b1_ref[...], 0.0)
    h2 = jnp.maximum(
        jnp.dot(w2_ref[...], h1, preferred_element_type=jnp.float32)
        + b2_ref[...], 0.0)
    h3 = jnp.maximum(
        jnp.dot(w3_ref[...], h2, preferred_element_type=jnp.float32)
        + b3_ref[...], 0.0)
    out_ref[...] = (
        jnp.sum(h3 * w4_ref[...], axis=0, keepdims=True) + b4_ref[...])


def _mlp(embT, numT_pad, W1eT, W1nT, b1c, W2T, b2c, W3T, b3c, w4c, b4):
    grid = (B // _BN,)
    full = lambda shape: pl.BlockSpec(shape, lambda i: (0, 0))
    return pl.pallas_call(
        _mlp_kernel,
        grid=grid,
        in_specs=[
            pl.BlockSpec((_FD, _BN), lambda i: (0, i)),
            pl.BlockSpec((16, _BN), lambda i: (0, i)),
            full((512, _FD)),
            full((512, 16)),
            full((512, 1)),
            full((256, 512)),
            full((256, 1)),
            full((128, 256)),
            full((128, 1)),
            full((128, 1)),
            full((1, 1)),
        ],
        out_specs=pl.BlockSpec((1, _BN), lambda i: (0, i)),
        out_shape=jax.ShapeDtypeStruct((1, B), jnp.float32),
        compiler_params=pltpu.CompilerParams(
            dimension_semantics=("arbitrary",),
        ),
    )(embT, numT_pad, W1eT, W1nT, b1c, W2T, b2c, W3T, b3c, w4c, b4)


def kernel(numerical_features, cat_features, tables, W1, b1, W2, b2, W3, b3,
           W4, b4):
    tt = jnp.transpose(tables, (0, 2, 1)).reshape(_FD, V)
    catT = cat_features.astype(jnp.int32).T

    embT = _sc_gather_t(tt, catT)

    numT_pad = jnp.pad(numerical_features.T, ((0, 16 - NUM), (0, 0)))
    W1eT = W1[:_FD].T
    W1nT = jnp.pad(W1[_FD:], ((0, 16 - NUM), (0, 0))).T
    outT = _mlp(embT, numT_pad, W1eT, W1nT,
                b1.reshape(-1, 1), W2.T, b2.reshape(-1, 1),
                W3.T, b3.reshape(-1, 1), W4.reshape(-1, 1),
                b4.reshape(1, 1))
    return outT.reshape(B, 1)


# speedup vs baseline: 1.3437x; 1.0279x over previous
import functools

import jax
import jax.numpy as jnp
from jax import lax
from jax.experimental import pallas as pl
from jax.experimental.pallas import tpu as pltpu
from jax.experimental.pallas import tpu_sc as plsc

B = 4096
F = 26
V = 100000
D = 32
NUM = 13

_NC = 2
_NS = 16
_NW = _NC * _NS
_FD = F * D
_ROWS_PER_W = _FD // _NW
_LANES = 16


def _sc_gather_t(tt, catT):
    mesh = plsc.VectorSubcoreMesh(core_axis_name="c", subcore_axis_name="s")

    @functools.partial(
        pl.kernel,
        out_type=jax.ShapeDtypeStruct((_FD, B), jnp.float32),
        mesh=mesh,
        scratch_types=[
            pltpu.VMEM((V,), jnp.float32),
            pltpu.VMEM((B,), jnp.int32),
            pltpu.VMEM((B,), jnp.float32),
            pltpu.VMEM((B,), jnp.float32),
            pltpu.SemaphoreType.DMA,
            pltpu.SemaphoreType.DMA,
            pltpu.SemaphoreType.DMA,
        ],
        compiler_params=pltpu.CompilerParams(needs_layout_passes=False),
    )
    def gather_kernel(tt_hbm, catT_hbm, out_hbm, row_v, idx_v, o0_v, o1_v,
                      row_sem, osem0, osem1):
        wid = lax.axis_index("s") * _NC + lax.axis_index("c")
        base = wid * _ROWS_PER_W
        osems = [osem0, osem1]
        obufs = [o0_v, o1_v]
        pltpu.sync_copy(catT_hbm.at[base // D], idx_v)
        out_copies = [None, None]
        for j in range(_ROWS_PER_W):
            r = base + j
            if j > 0:
                @pl.when((base + j) // D != (base + j - 1) // D)
                def _():
                    pltpu.sync_copy(catT_hbm.at[r // D], idx_v)
            pltpu.sync_copy(tt_hbm.at[r], row_v)
            if out_copies[j % 2] is not None:
                out_copies[j % 2].wait()
            ob = obufs[j % 2]

            @functools.partial(plsc.parallel_loop, 0, B // _LANES, unroll=1)
            def _(i):
                o = pl.multiple_of(i * _LANES, _LANES)
                iv = idx_v[pl.ds(o, _LANES)]
                ob[pl.ds(o, _LANES)] = plsc.load_gather(row_v, [iv])

            out_copies[j % 2] = pltpu.async_copy(
                ob, out_hbm.at[r], osems[j % 2])
        for c in out_copies:
            if c is not None:
                c.wait()

    return gather_kernel(tt, catT)


_BN = 512


def _mlp_kernel(embT_ref, numT_ref, w1e_ref, w1n_ref, b1_ref, w2_ref, b2_ref,
                w3_ref, b3_ref, w4_ref, b4_ref, out_ref):
    x1 = jnp.dot(w1e_ref[...], embT_ref[...],
                 preferred_element_type=jnp.float32)
    x1 = x1 + jnp.dot(w1n_ref[...], numT_ref[...],
                      preferred_element_type=jnp.float32)
    h1 = jnp.maximum(x1 + b1_ref[...], 0.0)
    h2 = jnp.maximum(
        jnp.dot(w2_ref[...], h1, preferred_element_type=jnp.float32)
        + b2_ref[...], 0.0)
    h3 = jnp.maximum(
        jnp.dot(w3_ref[...], h2, preferred_element_type=jnp.float32)
        + b3_ref[...], 0.0)
    out_ref[...] = (
        jnp.sum(h3 * w4_ref[...], axis=0, keepdims=True) + b4_ref[...])


def _mlp(embT, numT_pad, W1eT, W1nT, b1c, W2T, b2c, W3T, b3c, w4c, b4):
    grid = (B // _BN,)
    full = lambda shape: pl.BlockSpec(shape, lambda i: (0, 0))
    return pl.pallas_call(
        _mlp_kernel,
        grid=grid,
        in_specs=[
            pl.BlockSpec((_FD, _BN), lambda i: (0, i)),
            pl.BlockSpec((16, _BN), lambda i: (0, i)),
            full((512, _FD)),
            full((512, 16)),
            full((512, 1)),
            full((256, 512)),
            full((256, 1)),
            full((128, 256)),
            full((128, 1)),
            full((128, 1)),
            full((1, 1)),
        ],
        out_specs=pl.BlockSpec((1, _BN), lambda i: (0, i)),
        out_shape=jax.ShapeDtypeStruct((1, B), jnp.float32),
        compiler_params=pltpu.CompilerParams(
            dimension_semantics=("arbitrary",),
        ),
    )(embT, numT_pad, W1eT, W1nT, b1c, W2T, b2c, W3T, b3c, w4c, b4)


def kernel(numerical_features, cat_features, tables, W1, b1, W2, b2, W3, b3,
           W4, b4):
    tt = jnp.transpose(tables, (0, 2, 1)).reshape(_FD, V)
    catT = cat_features.astype(jnp.int32).T

    embT = _sc_gather_t(tt, catT)

    numT_pad = jnp.pad(numerical_features.T, ((0, 16 - NUM), (0, 0)))
    W1eT = W1[:_FD].T
    W1nT = jnp.pad(W1[_FD:], ((0, 16 - NUM), (0, 0))).T
    outT = _mlp(embT, numT_pad, W1eT, W1nT,
                b1.reshape(-1, 1), W2.T, b2.reshape(-1, 1),
                W3.T, b3.reshape(-1, 1), W4.reshape(-1, 1),
                b4.reshape(1, 1))
    return outT.reshape(B, 1)
